# Initial kernel scaffold; baseline (speedup 1.0000x reference)
import functools

import jax
import jax.numpy as jnp
from jax import lax
from jax.experimental import pallas as pl
from jax.experimental.pallas import tpu as pltpu
from jax.experimental.pallas import tpu_sc as plsc

_SHAPE = (4096, 11008)
_NUMEL = _SHAPE[0] * _SHAPE[1]
_NC, _NS = 2, 16
_NW = _NC * _NS
_REGION = _NUMEL // _NW
_BLK = 32768
_NBLK = _REGION // _BLK
_PC = 4096
_PCB = _PC + 16
_MROW = 96


def _vec_update(blk_v, bounce_i, bounce_f, idx16, val16, active, blk_base):
  iota = lax.iota(jnp.int32, 16)
  idx_m = jnp.where(active, idx16, -1)
  val_m = jnp.where(active, val16, 0.0)
  bounce_i[...] = idx_m
  up = plsc.load_gather(bounce_i, [jnp.minimum(iota + 1, 15)])
  dn = plsc.load_gather(bounce_i, [jnp.maximum(iota - 1, 0)])
  mask_last = active & ((iota == 15) | (up != idx_m))
  mask_first = active & ((iota == 0) | (dn != idx_m))
  t = plsc.cumsum(val_m)
  bounce_f[...] = t
  t_dn = plsc.load_gather(bounce_f, [jnp.maximum(iota - 1, 0)])
  t_dn = jnp.where(iota == 0, 0.0, t_dn)
  loc = idx_m - blk_base
  plsc.addupdate_scatter(blk_v, [loc], t, mask=mask_last)
  plsc.addupdate_scatter(blk_v, [loc], -t_dn, mask=mask_first)


def _sc_body(kt, flat_hbm, idx_hbm, val_hbm, meta_hbm, out_hbm,
             blk_v, idxc_v, valc_v, meta_v, bounce_i, bounce_f):
  cid = lax.axis_index("c")
  sid = lax.axis_index("s")
  wid = sid * _NC + cid
  region_base = wid * _REGION
  pltpu.sync_copy(meta_hbm.at[wid], meta_v)
  iota = lax.iota(jnp.int32, 16)

  def block_body(b, carry):
    off = (2 * b) // 16 * 16
    lane = 2 * b - off
    mvec = meta_v[pl.ds(off, 16)]
    sb = lax.reduce_max(jnp.where(iota == lane, mvec, 0), axes=(0,))
    eb = lax.reduce_max(jnp.where(iota == lane + 1, mvec, 0), axes=(0,))
    blk_base = region_base + b * _BLK
    pltpu.sync_copy(flat_hbm.at[pl.ds(blk_base, _BLK)], blk_v)
    nchunks = (eb - sb + _PC - 1) // _PC

    def chunk_body(ci, carry):
      cstart = sb + ci * _PC
      cend = jnp.minimum(eb, cstart + _PC)
      a = jnp.minimum((cstart // 8) * 8, kt - _PCB)
      pltpu.sync_copy(idx_hbm.at[pl.ds(a, _PCB)], idxc_v)
      pltpu.sync_copy(val_hbm.at[pl.ds(a, _PCB)], valc_v)
      nvec = (cend - a + 15) // 16

      def vec_body(v, carry):
        o = v * 16
        idx16 = idxc_v[pl.ds(o, 16)]
        val16 = valc_v[pl.ds(o, 16)]
        pos = a + o + iota
        active = (pos >= cstart) & (pos < cend)
        _vec_update(blk_v, bounce_i, bounce_f, idx16, val16, active, blk_base)
        return carry

      return lax.fori_loop(0, nvec, vec_body, carry)

    carry = lax.fori_loop(0, nchunks, chunk_body, carry)
    pltpu.sync_copy(blk_v, out_hbm.at[pl.ds(blk_base, _BLK)])
    return carry

  lax.fori_loop(0, _NBLK, block_body, 0)


def kernel(tensor, values, indices):
  flat = tensor.reshape(-1)
  k = values.shape[0]
  kt = ((k + _PCB + 7) // 8) * 8
  pad = kt - k
  idx_p = jnp.concatenate(
      [indices, jnp.full((pad,), _NUMEL - 1, dtype=jnp.int32)])
  val_p = jnp.concatenate([values.astype(jnp.float32),
                           jnp.zeros((pad,), dtype=jnp.float32)])
  boundaries = (jnp.arange(_NW * _NBLK + 1, dtype=jnp.int32) * _BLK)
  bs = jnp.searchsorted(idx_p, boundaries, side="left").astype(jnp.int32)
  inter = jnp.stack([bs[:-1], bs[1:]], axis=1).reshape(_NW, 2 * _NBLK)
  meta = jnp.zeros((_NW, _MROW), dtype=jnp.int32).at[:, : 2 * _NBLK].set(inter)

  mesh = plsc.VectorSubcoreMesh(
      core_axis_name="c", subcore_axis_name="s",
      num_cores=_NC, num_subcores=_NS)
  run = pl.kernel(
      functools.partial(_sc_body, kt),
      out_type=jax.ShapeDtypeStruct((_NUMEL,), jnp.float32),
      mesh=mesh,
      scratch_types=[
          pltpu.VMEM((_BLK,), jnp.float32),
          pltpu.VMEM((_PCB,), jnp.int32),
          pltpu.VMEM((_PCB,), jnp.float32),
          pltpu.VMEM((_MROW,), jnp.int32),
          pltpu.VMEM((16,), jnp.int32),
          pltpu.VMEM((16,), jnp.float32),
      ],
  )
  out = run(flat, idx_p, val_p, meta)
  return out.reshape(_SHAPE)


# speedup vs baseline: 3.0778x; 3.0778x over previous
import functools

import jax
import jax.numpy as jnp
from jax import lax
from jax.experimental import pallas as pl
from jax.experimental.pallas import tpu as pltpu
from jax.experimental.pallas import tpu_sc as plsc

_SHAPE = (4096, 11008)
_NUMEL = _SHAPE[0] * _SHAPE[1]
_NC, _NS = 2, 16
_NW = _NC * _NS
_REGION = _NUMEL // _NW
_BLK = 32768
_NBLK = _REGION // _BLK
_PC = 4096
_PCB = _PC + 16
_MROW = _NBLK * 16


def _vec_update(blk_v, bounce_i, bounce_f, idx16, val16, active, blk_base):
  iota = lax.iota(jnp.int32, 16)
  idx_m = jnp.where(active, idx16, -1)
  val_m = jnp.where(active, val16, 0.0)
  bounce_i[...] = idx_m
  up = plsc.load_gather(bounce_i, [jnp.minimum(iota + 1, 15)])
  dn = plsc.load_gather(bounce_i, [jnp.maximum(iota - 1, 0)])
  mask_last = active & ((iota == 15) | (up != idx_m))
  mask_first = active & ((iota == 0) | (dn != idx_m))
  t = plsc.cumsum(val_m)
  bounce_f[...] = t
  t_dn = plsc.load_gather(bounce_f, [jnp.maximum(iota - 1, 0)])
  t_dn = jnp.where(iota == 0, 0.0, t_dn)
  loc = idx_m - blk_base
  plsc.addupdate_scatter(blk_v, [loc], t, mask=mask_last)
  plsc.addupdate_scatter(blk_v, [loc], -t_dn, mask=mask_first)


def _sc_body(kt, flat_hbm, idx_hbm, val_hbm, meta_hbm, out_hbm,
             blk_v, idxc_v, valc_v, meta_v, bounce_i, bounce_f):
  cid = lax.axis_index("c")
  sid = lax.axis_index("s")
  wid = sid * _NC + cid
  region_base = wid * _REGION
  pltpu.sync_copy(meta_hbm.at[wid], meta_v)
  iota = lax.iota(jnp.int32, 16)

  def block_body(b, carry):
    mvec = meta_v[pl.ds(b * 16, 16)]
    sb = mvec[0]
    eb = mvec[1]
    blk_base = region_base + b * _BLK
    pltpu.sync_copy(flat_hbm.at[pl.ds(blk_base, _BLK)], blk_v)
    nchunks = (eb - sb + _PC - 1) // _PC

    def chunk_body(ci, carry):
      cstart = sb + ci * _PC
      cend = jnp.minimum(eb, cstart + _PC)
      a = jnp.minimum((cstart // 8) * 8, kt - _PCB)
      pltpu.sync_copy(idx_hbm.at[pl.ds(a, _PCB)], idxc_v)
      pltpu.sync_copy(val_hbm.at[pl.ds(a, _PCB)], valc_v)
      nvec = (cend - a + 15) // 16

      def vec_body(v, carry):
        o = v * 16
        idx16 = idxc_v[pl.ds(o, 16)]
        val16 = valc_v[pl.ds(o, 16)]
        pos = a + o + iota
        active = (pos >= cstart) & (pos < cend)
        _vec_update(blk_v, bounce_i, bounce_f, idx16, val16, active, blk_base)
        return carry

      return lax.fori_loop(0, nvec, vec_body, carry)

    carry = lax.fori_loop(0, nchunks, chunk_body, carry)
    pltpu.sync_copy(blk_v, out_hbm.at[pl.ds(blk_base, _BLK)])
    return carry

  lax.fori_loop(0, _NBLK, block_body, 0)


def kernel(tensor, values, indices):
  flat = tensor.reshape(-1)
  k = values.shape[0]
  kt = ((k + _PCB + 7) // 8) * 8
  pad = kt - k
  idx_p = jnp.concatenate(
      [indices, jnp.full((pad,), _NUMEL - 1, dtype=jnp.int32)])
  val_p = jnp.concatenate([values.astype(jnp.float32),
                           jnp.zeros((pad,), dtype=jnp.float32)])
  boundaries = (jnp.arange(_NW * _NBLK + 1, dtype=jnp.int32) * _BLK)
  bs = jnp.searchsorted(idx_p, boundaries, side="left").astype(jnp.int32)
  inter = jnp.stack([bs[:-1], bs[1:]], axis=1).reshape(_NW, _NBLK, 2)
  meta = (jnp.zeros((_NW, _NBLK, 16), dtype=jnp.int32)
          .at[:, :, :2].set(inter).reshape(_NW, _MROW))

  mesh = plsc.VectorSubcoreMesh(
      core_axis_name="c", subcore_axis_name="s",
      num_cores=_NC, num_subcores=_NS)
  run = pl.kernel(
      functools.partial(_sc_body, kt),
      out_type=jax.ShapeDtypeStruct((_NUMEL,), jnp.float32),
      mesh=mesh,
      compiler_params=pltpu.CompilerParams(needs_layout_passes=False),
      scratch_types=[
          pltpu.VMEM((_BLK,), jnp.float32),
          pltpu.VMEM((_PCB,), jnp.int32),
          pltpu.VMEM((_PCB,), jnp.float32),
          pltpu.VMEM((_MROW,), jnp.int32),
          pltpu.VMEM((16,), jnp.int32),
          pltpu.VMEM((16,), jnp.float32),
      ],
  )
  out = run(flat, idx_p, val_p, meta)
  return out.reshape(_SHAPE)


# speedup vs baseline: 3.6587x; 1.1888x over previous
import functools

import jax
import jax.numpy as jnp
from jax import lax
from jax.experimental import pallas as pl
from jax.experimental.pallas import tpu as pltpu
from jax.experimental.pallas import tpu_sc as plsc

_SHAPE = (4096, 11008)
_NUMEL = _SHAPE[0] * _SHAPE[1]
_NC, _NS = 2, 16
_NW = _NC * _NS
_REGION = _NUMEL // _NW
_BLK = 32768
_NBLK = _REGION // _BLK
_PC = 1536
_PCB = _PC + 16
_MROW = _NBLK * 16


def _vec_update(blk_v, bounce_i, bounce_f, idx16, val16, active, blk_base):
  iota = lax.iota(jnp.int32, 16)
  idx_m = jnp.where(active, idx16, -1)
  val_m = jnp.where(active, val16, 0.0)
  bounce_i[...] = idx_m
  up = plsc.load_gather(bounce_i, [jnp.minimum(iota + 1, 15)])
  dn = plsc.load_gather(bounce_i, [jnp.maximum(iota - 1, 0)])
  mask_last = active & ((iota == 15) | (up != idx_m))
  mask_first = active & ((iota == 0) | (dn != idx_m))
  t = plsc.cumsum(val_m)
  bounce_f[...] = t
  t_dn = plsc.load_gather(bounce_f, [jnp.maximum(iota - 1, 0)])
  t_dn = jnp.where(iota == 0, 0.0, t_dn)
  loc = idx_m - blk_base
  plsc.addupdate_scatter(blk_v, [loc], t, mask=mask_last)
  plsc.addupdate_scatter(blk_v, [loc], -t_dn, mask=mask_first)


class _BufSet:
  def __init__(self, blk, pi, pv, sem_in, sem_out, sem_pi, sem_pv):
    self.blk, self.pi, self.pv = blk, pi, pv
    self.sem_in, self.sem_out = sem_in, sem_out
    self.sem_pi, self.sem_pv = sem_pi, sem_pv


def _sc_body(kt, flat_hbm, idx_hbm, val_hbm, meta_hbm, out_hbm,
             blk0, blk1, pi0, pv0, pi1, pv1, ri, rv, meta_v,
             bounce_i, bounce_f,
             sem_in0, sem_in1, sem_out0, sem_out1,
             sem_pi0, sem_pi1, sem_pv0, sem_pv1):
  cid = lax.axis_index("c")
  sid = lax.axis_index("s")
  wid = sid * _NC + cid
  region_base = wid * _REGION
  pltpu.sync_copy(meta_hbm.at[wid], meta_v)

  set0 = _BufSet(blk0, pi0, pv0, sem_in0, sem_out0, sem_pi0, sem_pv0)
  set1 = _BufSet(blk1, pi1, pv1, sem_in1, sem_out1, sem_pi1, sem_pv1)

  def get_se(b):
    mvec = meta_v[pl.ds(b * 16, 16)]
    return mvec[0], mvec[1]

  def pair_a(p):
    return jnp.minimum((p // 8) * 8, kt - _PCB)

  def in_copy(b, s):
    return pltpu.make_async_copy(
        flat_hbm.at[pl.ds(region_base + b * _BLK, _BLK)], s.blk, s.sem_in)

  def out_copy(b, s):
    return pltpu.make_async_copy(
        s.blk, out_hbm.at[pl.ds(region_base + b * _BLK, _BLK)], s.sem_out)

  def pair_copies(a, s):
    return (pltpu.make_async_copy(idx_hbm.at[pl.ds(a, _PCB)], s.pi, s.sem_pi),
            pltpu.make_async_copy(val_hbm.at[pl.ds(a, _PCB)], s.pv, s.sem_pv))

  def issue_front(b, s):
    in_copy(b, s).start()
    sb, _ = get_se(b)
    ci, cv = pair_copies(pair_a(sb), s)
    ci.start()
    cv.start()

  def consume(blk, idxb, valb, a, cstart, cend, blk_base):
    nvec = (cend - a + 15) // 16

    def vec_body(v, carry):
      o = v * 16
      idx16 = idxb[pl.ds(o, 16)]
      val16 = valb[pl.ds(o, 16)]
      pos = a + o + lax.iota(jnp.int32, 16)
      active = (pos >= cstart) & (pos < cend)
      _vec_update(blk, bounce_i, bounce_f, idx16, val16, active, blk_base)
      return carry

    lax.fori_loop(0, nvec, vec_body, 0)

  def half(b, mine, other):
    sb, eb = get_se(b)

    @pl.when(b >= 1)
    def _():
      out_copy(b - 1, other).wait()

    @pl.when(b + 1 < _NBLK)
    def _():
      issue_front(b + 1, other)

    ci, cv = pair_copies(0, mine)
    ci.wait()
    cv.wait()
    in_copy(b, mine).wait()

    blk_base = region_base + b * _BLK
    a0 = pair_a(sb)
    consume(mine.blk, mine.pi, mine.pv, a0, sb, jnp.minimum(eb, sb + _PC),
            blk_base)
    nchunks = (eb - sb + _PC - 1) // _PC

    def chunk_body(c, carry):
      cstart = sb + c * _PC
      cend = jnp.minimum(eb, cstart + _PC)
      ac = pair_a(cstart)
      pltpu.sync_copy(idx_hbm.at[pl.ds(ac, _PCB)], ri)
      pltpu.sync_copy(val_hbm.at[pl.ds(ac, _PCB)], rv)
      consume(mine.blk, ri, rv, ac, cstart, cend, blk_base)
      return carry

    lax.fori_loop(1, jnp.maximum(nchunks, 1), chunk_body, 0)
    out_copy(b, mine).start()

  issue_front(0, set0)

  def pair_step(g, carry):
    b0 = 2 * g
    half(b0, set0, set1)

    @pl.when(b0 + 1 < _NBLK)
    def _():
      half(b0 + 1, set1, set0)

    return carry

  lax.fori_loop(0, (_NBLK + 1) // 2, pair_step, 0)
  out_copy(_NBLK - 1, set0 if (_NBLK - 1) % 2 == 0 else set1).wait()


def kernel(tensor, values, indices):
  flat = tensor.reshape(-1)
  k = values.shape[0]
  kt = ((k + _PCB + 7) // 8) * 8
  pad = kt - k
  idx_p = jnp.concatenate(
      [indices, jnp.full((pad,), _NUMEL - 1, dtype=jnp.int32)])
  val_p = jnp.concatenate([values.astype(jnp.float32),
                           jnp.zeros((pad,), dtype=jnp.float32)])
  boundaries = (jnp.arange(_NW * _NBLK + 1, dtype=jnp.int32) * _BLK)
  bs = jnp.searchsorted(idx_p, boundaries, side="left").astype(jnp.int32)
  inter = jnp.stack([bs[:-1], bs[1:]], axis=1).reshape(_NW, _NBLK, 2)
  meta = (jnp.zeros((_NW, _NBLK, 16), dtype=jnp.int32)
          .at[:, :, :2].set(inter).reshape(_NW, _MROW))

  mesh = plsc.VectorSubcoreMesh(
      core_axis_name="c", subcore_axis_name="s",
      num_cores=_NC, num_subcores=_NS)
  run = pl.kernel(
      functools.partial(_sc_body, kt),
      out_type=jax.ShapeDtypeStruct((_NUMEL,), jnp.float32),
      mesh=mesh,
      compiler_params=pltpu.CompilerParams(needs_layout_passes=False),
      scratch_types=[
          pltpu.VMEM((_BLK,), jnp.float32),
          pltpu.VMEM((_BLK,), jnp.float32),
          pltpu.VMEM((_PCB,), jnp.int32),
          pltpu.VMEM((_PCB,), jnp.float32),
          pltpu.VMEM((_PCB,), jnp.int32),
          pltpu.VMEM((_PCB,), jnp.float32),
          pltpu.VMEM((_PCB,), jnp.int32),
          pltpu.VMEM((_PCB,), jnp.float32),
          pltpu.VMEM((_MROW,), jnp.int32),
          pltpu.VMEM((16,), jnp.int32),
          pltpu.VMEM((16,), jnp.float32),
          pltpu.SemaphoreType.DMA,
          pltpu.SemaphoreType.DMA,
          pltpu.SemaphoreType.DMA,
          pltpu.SemaphoreType.DMA,
          pltpu.SemaphoreType.DMA,
          pltpu.SemaphoreType.DMA,
          pltpu.SemaphoreType.DMA,
          pltpu.SemaphoreType.DMA,
      ],
  )
  out = run(flat, idx_p, val_p, meta)
  return out.reshape(_SHAPE)
